# baseline (speedup 1.0000x reference)
import jax
import jax.numpy as jnp
from jax.experimental import pallas as pl
from jax.experimental.pallas import tpu as pltpu

_K = 4
_C = 2560
_N = 10000


def _agg_body(a_ref, x_ref, wne_ref, bne_ref, wen_ref,
              h_ref, g_ref, sums_ref, counts_ref):
    i = pl.program_id(0)
    col0 = i * _C
    lane = jax.lax.broadcasted_iota(jnp.int32, (1, _C), 1)
    A = jnp.where(col0 + lane < _N, a_ref[:], 0.0)
    row = jax.lax.broadcasted_iota(jnp.int32, (_C, 1), 0)
    X = jnp.where(col0 + row < _N, x_ref[:], 0.0)

    @pl.when(i == 0)
    def _init():
        sums_ref[:] = jnp.zeros_like(sums_ref)
        counts_ref[:] = jnp.zeros_like(counts_ref)

    Ab = A.astype(jnp.bfloat16)
    Xb = X.astype(jnp.bfloat16)
    sums_ref[:128] += jax.lax.dot_general(
        Ab[:128], Xb, (((1,), (0,)), ((), ())),
        preferred_element_type=jnp.float32)
    sums_ref[128:] += jax.lax.dot_general(
        Ab[128:], Xb, (((1,), (0,)), ((), ())),
        preferred_element_type=jnp.float32)
    counts_ref[:] += jnp.sum(A, axis=1, keepdims=True)

    @pl.when(i == _K - 1)
    def _finish():
        mean = sums_ref[:] / jnp.maximum(counts_ref[:], 1.0)
        H = jnp.dot(mean, wne_ref[:],
                    preferred_element_type=jnp.float32) + bne_ref[:]
        h_ref[:] = H
        g_ref[:] = jnp.dot(H, wen_ref[:], preferred_element_type=jnp.float32)


def _prop_body(a_ref, x_ref, g_ref, ben_ref, y_ref):
    Ab = a_ref[:].astype(jnp.bfloat16)
    G = g_ref[:]
    Gb = G.astype(jnp.bfloat16)
    y1 = jax.lax.dot_general(Ab[:128], Gb[:128], (((0,), (0,)), ((), ())),
                             preferred_element_type=jnp.float32)
    y2 = jax.lax.dot_general(Ab[128:], Gb[128:], (((0,), (0,)), ((), ())),
                             preferred_element_type=jnp.float32)
    y_ref[:] = (y1 + y2) + ben_ref[:] + x_ref[:]


def kernel(node_features, incidence_matrix, W_ne, b_ne, W_en, b_en):
    n_edges = incidence_matrix.shape[0]
    n_nodes, in_ch = node_features.shape
    out_ch = W_ne.shape[1]

    h, g = pl.pallas_call(
        _agg_body,
        grid=(_K,),
        in_specs=[
            pl.BlockSpec((n_edges, _C), lambda i: (0, i)),
            pl.BlockSpec((_C, in_ch), lambda i: (i, 0)),
            pl.BlockSpec((in_ch, out_ch), lambda i: (0, 0)),
            pl.BlockSpec((1, out_ch), lambda i: (0, 0)),
            pl.BlockSpec((out_ch, out_ch), lambda i: (0, 0)),
        ],
        out_specs=(
            pl.BlockSpec((n_edges, out_ch), lambda i: (0, 0)),
            pl.BlockSpec((n_edges, out_ch), lambda i: (0, 0)),
        ),
        out_shape=(
            jax.ShapeDtypeStruct((n_edges, out_ch), jnp.float32),
            jax.ShapeDtypeStruct((n_edges, out_ch), jnp.float32),
        ),
        scratch_shapes=[
            pltpu.VMEM((n_edges, out_ch), jnp.float32),
            pltpu.VMEM((n_edges, out_ch), jnp.float32),
        ],
    )(incidence_matrix, node_features, W_ne, b_ne.reshape(1, -1), W_en)

    y = pl.pallas_call(
        _prop_body,
        grid=(_K,),
        in_specs=[
            pl.BlockSpec((n_edges, _C), lambda i: (0, i)),
            pl.BlockSpec((_C, in_ch), lambda i: (i, 0)),
            pl.BlockSpec((n_edges, out_ch), lambda i: (0, 0)),
            pl.BlockSpec((1, out_ch), lambda i: (0, 0)),
        ],
        out_specs=pl.BlockSpec((_C, out_ch), lambda i: (i, 0)),
        out_shape=jax.ShapeDtypeStruct((n_nodes, out_ch), jnp.float32),
    )(incidence_matrix, node_features, g, b_en.reshape(1, -1))

    attention_weights = jnp.ones((n_edges,), dtype=jnp.float32)
    return (y, h, attention_weights)


# speedup vs baseline: 1.1284x; 1.1284x over previous
import jax
import jax.numpy as jnp
from jax.experimental import pallas as pl
from jax.experimental.pallas import tpu as pltpu

_N = 10000
_E = 256
_F = 128
_RK = 4
_R = _E // _RK
_YCH = (2560, 2560, 2560, 2320)


def _body(a_hbm, x_hbm, wne_ref, bne_ref, wen_ref, ben_ref,
          y_hbm, h_ref,
          ab_vmem, x_vmem, y_scr, sem_x, sem_a, sem_y,
          a_land):
    cp_x = pltpu.make_async_copy(x_hbm, x_vmem, sem_x)
    cp_x.start()
    for c in range(_RK):
        pltpu.make_async_copy(
            a_hbm.at[pl.ds(c * _R, _R), :],
            a_land.at[pl.ds(c * _R, _R), :],
            sem_a.at[c],
        ).start()
    cp_x.wait()
    Xb = x_vmem[:].astype(jnp.bfloat16)

    s_parts = []
    c_parts = []
    for c in range(_RK):
        pltpu.make_async_copy(
            a_hbm.at[pl.ds(c * _R, _R), :],
            a_land.at[pl.ds(c * _R, _R), :],
            sem_a.at[c],
        ).wait()
        Ac = a_land[pl.ds(c * _R, _R), :]
        Acb = Ac.astype(jnp.bfloat16)
        ab_vmem[pl.ds(c * _R, _R), :] = Acb
        s_parts.append(jax.lax.dot_general(
            Acb, Xb, (((1,), (0,)), ((), ())),
            preferred_element_type=jnp.float32))
        c_parts.append(jnp.sum(Ac, axis=1, keepdims=True))

    sums = jnp.concatenate(s_parts, axis=0)
    counts = jnp.concatenate(c_parts, axis=0)
    mean = sums / jnp.maximum(counts, 1.0)
    H = jnp.dot(mean, wne_ref[:], preferred_element_type=jnp.float32) + bne_ref[:]
    h_ref[:] = H
    G = jnp.dot(H, wen_ref[:], preferred_element_type=jnp.float32)
    Gb = G.astype(jnp.bfloat16)

    Ab = ab_vmem[:]
    y1 = jax.lax.dot_general(Ab[:128], Gb[:128], (((0,), (0,)), ((), ())),
                             preferred_element_type=jnp.float32)
    y2 = jax.lax.dot_general(Ab[128:], Gb[128:], (((0,), (0,)), ((), ())),
                             preferred_element_type=jnp.float32)
    y_scr[:] = (y1 + y2) + ben_ref[:] + x_vmem[:]

    off = 0
    for c, rows in enumerate(_YCH):
        pltpu.make_async_copy(
            y_scr.at[pl.ds(off, rows), :],
            y_hbm.at[pl.ds(off, rows), :],
            sem_y.at[c],
        ).start()
        off += rows
    off = 0
    for c, rows in enumerate(_YCH):
        pltpu.make_async_copy(
            y_scr.at[pl.ds(off, rows), :],
            y_hbm.at[pl.ds(off, rows), :],
            sem_y.at[c],
        ).wait()
        off += rows


def kernel(node_features, incidence_matrix, W_ne, b_ne, W_en, b_en):
    n_edges = incidence_matrix.shape[0]
    n_nodes, in_ch = node_features.shape
    out_ch = W_ne.shape[1]

    y, h = pl.pallas_call(
        _body,
        in_specs=[
            pl.BlockSpec(memory_space=pl.ANY),
            pl.BlockSpec(memory_space=pl.ANY),
            pl.BlockSpec((in_ch, out_ch), lambda: (0, 0)),
            pl.BlockSpec((1, out_ch), lambda: (0, 0)),
            pl.BlockSpec((out_ch, out_ch), lambda: (0, 0)),
            pl.BlockSpec((1, out_ch), lambda: (0, 0)),
        ],
        out_specs=(
            pl.BlockSpec(memory_space=pl.ANY),
            pl.BlockSpec((n_edges, out_ch), lambda: (0, 0)),
        ),
        out_shape=(
            jax.ShapeDtypeStruct((n_nodes, out_ch), jnp.float32),
            jax.ShapeDtypeStruct((n_edges, out_ch), jnp.float32),
        ),
        scratch_shapes=[
            pltpu.VMEM((_E, _N), jnp.bfloat16),
            pltpu.VMEM((_N, _F), jnp.float32),
            pltpu.VMEM((_N, _F), jnp.float32),
            pltpu.SemaphoreType.DMA,
            pltpu.SemaphoreType.DMA((_RK,)),
            pltpu.SemaphoreType.DMA((len(_YCH),)),
            pltpu.VMEM((_E, _N), jnp.float32),
        ],
    )(incidence_matrix, node_features, W_ne, b_ne.reshape(1, -1),
      W_en, b_en.reshape(1, -1))
    attention_weights = jnp.ones((n_edges,), dtype=jnp.float32)
    return (y, h, attention_weights)


# speedup vs baseline: 1.2231x; 1.0839x over previous
import jax
import jax.numpy as jnp
from jax.experimental import pallas as pl


def _body(a_ref, x_ref, wne_ref, bne_ref, wen_ref, ben_ref, y_ref, h_ref):
    Ab = a_ref[:].astype(jnp.bfloat16)
    Xb = x_ref[:].astype(jnp.bfloat16)

    s1 = jax.lax.dot_general(Ab[:128], Xb, (((1,), (0,)), ((), ())),
                             preferred_element_type=jnp.float32)
    s2 = jax.lax.dot_general(Ab[128:], Xb, (((1,), (0,)), ((), ())),
                             preferred_element_type=jnp.float32)
    sums = jnp.concatenate([s1, s2], axis=0)
    counts = jnp.sum(Ab, axis=1, keepdims=True, dtype=jnp.float32)

    mean = sums / jnp.maximum(counts, 1.0)
    H = jnp.dot(mean, wne_ref[:], preferred_element_type=jnp.float32) + bne_ref[:]
    h_ref[:] = H
    G = jnp.dot(H, wen_ref[:], preferred_element_type=jnp.float32)
    Gb = G.astype(jnp.bfloat16)

    y1 = jax.lax.dot_general(Ab[:128], Gb[:128], (((0,), (0,)), ((), ())),
                             preferred_element_type=jnp.float32)
    y2 = jax.lax.dot_general(Ab[128:], Gb[128:], (((0,), (0,)), ((), ())),
                             preferred_element_type=jnp.float32)
    y_ref[:] = (y1 + y2) + ben_ref[:] + x_ref[:]


def kernel(node_features, incidence_matrix, W_ne, b_ne, W_en, b_en):
    n_edges = incidence_matrix.shape[0]
    n_nodes, in_ch = node_features.shape
    out_ch = W_ne.shape[1]
    y, h = pl.pallas_call(
        _body,
        out_shape=(
            jax.ShapeDtypeStruct((n_nodes, out_ch), jnp.float32),
            jax.ShapeDtypeStruct((n_edges, out_ch), jnp.float32),
        ),
    )(incidence_matrix, node_features, W_ne, b_ne.reshape(1, -1),
      W_en, b_en.reshape(1, -1))
    attention_weights = jnp.ones((n_edges,), dtype=jnp.float32)
    return (y, h, attention_weights)
